# baseline (speedup 1.0000x reference)
import functools

import jax
import jax.numpy as jnp
from jax import lax
from jax.experimental import pallas as pl
from jax.experimental.pallas import tpu as pltpu
from jax.experimental.pallas import tpu_sc as plsc

_MESH = plsc.ScalarSubcoreMesh(axis_name="c", num_cores=1)


@functools.partial(
    pl.kernel,
    mesh=_MESH,
    out_type=[
        jax.ShapeDtypeStruct((1, 3), jnp.float32),
        jax.ShapeDtypeStruct((1, 3), jnp.float32),
    ],
    scratch_types=[
        pltpu.SMEM((1,), jnp.int32),
        pltpu.SMEM((1, 3), jnp.float32),
        pltpu.SMEM((1, 3), jnp.float32),
        pltpu.SemaphoreType.DMA,
    ],
)
def _sc_lookup(idx_hbm, rot_hbm, trans_hbm, theta_hbm, rho_hbm,
               idx_s, theta_s, rho_s, sem):
    pltpu.sync_copy(idx_hbm, idx_s)
    i = idx_s[0]
    a = pltpu.make_async_copy(rot_hbm.at[pl.ds(i, 1)], theta_s, sem)
    b = pltpu.make_async_copy(trans_hbm.at[pl.ds(i, 1)], rho_s, sem)
    a.start()
    b.start()
    a.wait()
    b.wait()
    pltpu.sync_copy(theta_s, theta_hbm)
    pltpu.sync_copy(rho_s, rho_hbm)


def kernel(view_ids, rot_weight, trans_weight):
    idx = view_ids[:1].astype(jnp.int32)
    theta, rho = _sc_lookup(idx, rot_weight, trans_weight)
    return (theta, rho)


# speedup vs baseline: 1.0341x; 1.0341x over previous
import functools

import jax
import jax.numpy as jnp
from jax import lax
from jax.experimental import pallas as pl
from jax.experimental.pallas import tpu as pltpu
from jax.experimental.pallas import tpu_sc as plsc

_MESH = plsc.ScalarSubcoreMesh(axis_name="c", num_cores=1)


@functools.partial(
    pl.kernel,
    mesh=_MESH,
    out_type=[
        jax.ShapeDtypeStruct((1, 3), jnp.float32),
        jax.ShapeDtypeStruct((1, 3), jnp.float32),
    ],
    scratch_types=[
        pltpu.SMEM((1,), jnp.int32),
        pltpu.SemaphoreType.DMA,
    ],
)
def _sc_lookup(idx_hbm, rot_hbm, trans_hbm, theta_hbm, rho_hbm,
               idx_s, sem):
    pltpu.sync_copy(idx_hbm, idx_s)
    i = idx_s[0]
    a = pltpu.make_async_copy(rot_hbm.at[pl.ds(i, 1)], theta_hbm, sem)
    b = pltpu.make_async_copy(trans_hbm.at[pl.ds(i, 1)], rho_hbm, sem)
    a.start()
    b.start()
    a.wait()
    b.wait()


def kernel(view_ids, rot_weight, trans_weight):
    idx = view_ids[:1].astype(jnp.int32)
    theta, rho = _sc_lookup(idx, rot_weight, trans_weight)
    return (theta, rho)


# speedup vs baseline: 1.0415x; 1.0071x over previous
import functools

import jax
import jax.numpy as jnp
from jax.experimental import pallas as pl
from jax.experimental.pallas import tpu as pltpu
from jax.experimental.pallas import tpu_sc as plsc

_MESH = plsc.ScalarSubcoreMesh(axis_name="c", num_cores=1)


@functools.partial(
    pl.kernel,
    mesh=_MESH,
    out_type=[
        jax.ShapeDtypeStruct((1, 3), jnp.float32),
        jax.ShapeDtypeStruct((1, 3), jnp.float32),
    ],
    scratch_types=[
        pltpu.SMEM((1,), jnp.int32),
        pltpu.SemaphoreType.DMA,
    ],
)
def _sc_lookup(idx_hbm, rot_hbm, trans_hbm, theta_hbm, rho_hbm,
               idx_s, sem):
    pltpu.sync_copy(idx_hbm, idx_s)
    i = idx_s[0]
    a = pltpu.make_async_copy(rot_hbm.at[pl.ds(i, 1)], theta_hbm, sem)
    b = pltpu.make_async_copy(trans_hbm.at[pl.ds(i, 1)], rho_hbm, sem)
    a.start()
    b.start()
    a.wait()
    b.wait()


def kernel(view_ids, rot_weight, trans_weight):
    idx = view_ids[:1].astype(jnp.int32)
    theta, rho = _sc_lookup(idx, rot_weight, trans_weight)
    return (theta, rho)
